# baseline (speedup 1.0000x reference)
import functools

import jax
import jax.numpy as jnp
from jax import lax
from jax.experimental import pallas as pl
from jax.experimental.pallas import tpu as pltpu
from jax.experimental.pallas import tpu_sc as plsc

N_NODES = 50000
N_EDGES = 800000
D_IN = 128
D_H = 64
D_A = 80

NC = 2
NS = 16
HALF = N_NODES // NC
STRIPE = 1568
ROWS = NS * STRIPE
TRASH = HALF
G = 32
RT = N_EDGES // G
NCHU = 1564
KI = 4
KR = 2

_sc_mesh = plsc.VectorSubcoreMesh(core_axis_name="c", subcore_axis_name="s")
_sc_params = pltpu.CompilerParams(use_tc_tiling_on_sc=False)


def _localize(base, ldst_v, slot, voff):
    for q in range(G // 16):
        d = ldst_v[slot, pl.ds(q * 16, 16)] + voff
        m = (d >= base) & (d < base + HALF)
        ldst_v[slot, pl.ds(q * 16, 16)] = jnp.where(m, d - base, TRASH)


def _make_agg():
    scratch = [
        pltpu.VMEM((KI, G), jnp.int32),
        pltpu.VMEM((KI, G), jnp.int32),
        pltpu.VMEM((KR, G, D_A), jnp.float32),
        pltpu.VMEM_SHARED((ROWS, D_A), jnp.float32),
        pltpu.SemaphoreType.DMA,
        pltpu.SemaphoreType.DMA,
        pltpu.SemaphoreType.DMA,
        pltpu.SemaphoreType.DMA,
    ]

    def body(h_hbm, src2, dst2, z80, agg_out,
             src_v, ldst_v, rows_v, acc_sh, sem_ia, sem_ib, sem_g, sem_s):
        c = lax.axis_index("c")
        s = lax.axis_index("s")
        base = c * HALF
        start = s * (RT // NS) + jnp.minimum(s, RT % NS)
        nrows = (RT // NS) + jnp.where(s < RT % NS, 1, 0)

        def fire_idx(j, sem):
            r = jnp.minimum(start + j, RT - 1)
            slot = lax.rem(j, KI)
            pltpu.async_copy(src2.at[r], src_v.at[slot], sem)
            pltpu.async_copy(dst2.at[r], ldst_v.at[slot], sem)

        def drain_idx(j, sem):
            slot = lax.rem(j, KI)
            pltpu.make_async_copy(src2.at[0], src_v.at[slot], sem).wait()
            pltpu.make_async_copy(dst2.at[0], ldst_v.at[slot], sem).wait()

        def localize(j):
            voff = jnp.where(j < nrows, 0, N_NODES)
            _localize(base, ldst_v, lax.rem(j, KI), voff)

        def fire_gather(j):
            pltpu.async_copy(h_hbm.at[src_v.at[lax.rem(j, KI)]],
                             rows_v.at[lax.rem(j, KR)], sem_g)

        def wait_gather(j):
            pltpu.make_async_copy(h_hbm.at[src_v.at[lax.rem(j, KI)]],
                                  rows_v.at[lax.rem(j, KR)], sem_g).wait()

        def fire_scatter(j):
            pltpu.async_copy(rows_v.at[lax.rem(j, KR)],
                             acc_sh.at[ldst_v.at[lax.rem(j, KI)]],
                             sem_s, add=True)

        def wait_scatter(j):
            pltpu.make_async_copy(rows_v.at[lax.rem(j, KR)],
                                  acc_sh.at[ldst_v.at[lax.rem(j, KI)]],
                                  sem_s).wait()

        def isem(j):
            return sem_ia if j % 2 == 0 else sem_ib

        pltpu.sync_copy(z80, acc_sh.at[pl.ds(s * STRIPE, STRIPE)])
        plsc.subcore_barrier()
        fire_idx(0, sem_ia)
        drain_idx(0, sem_ia)
        localize(0)
        fire_gather(0)
        fire_idx(1, sem_ib)
        fire_idx(2, sem_ia)

        def cu(j, par):
            wait_gather(j)
            fire_scatter(j)

            @pl.when(j >= 1)
            def _():
                wait_scatter(j - 1)

            @pl.when(j + 1 < NCHU)
            def _():
                drain_idx(j + 1, isem(par + 1))
                localize(j + 1)
                fire_gather(j + 1)

            @pl.when(j + 3 < NCHU)
            def _():
                fire_idx(j + 3, isem(par + 1))

        def pair_body(i, carry):
            cu(2 * i, 0)
            cu(2 * i + 1, 1)
            return carry

        lax.fori_loop(0, NCHU // 2, pair_body, 0)
        wait_scatter(NCHU - 1)

        plsc.subcore_barrier()
        pltpu.sync_copy(acc_sh.at[pl.ds(s * STRIPE, STRIPE)],
                        agg_out.at[c].at[pl.ds(s * STRIPE, STRIPE)])

    return pl.kernel(
        body,
        out_type=jax.ShapeDtypeStruct((NC, ROWS, D_A), jnp.float32),
        mesh=_sc_mesh,
        scratch_types=scratch,
        compiler_params=_sc_params,
    )


_sc_agg = _make_agg()


_R = 2000
_PREC = lax.Precision.HIGHEST


def _mlp_in_body(x_ref, w_ref, b_ref, o_ref):
    y = jnp.maximum(
        jnp.dot(x_ref[...], w_ref[...], preferred_element_type=jnp.float32,
                precision=_PREC) + b_ref[...], 0.0)
    o_ref[...] = jnp.concatenate(
        [y, jnp.ones((y.shape[0], D_A - D_H), jnp.float32)], axis=1)


def _mlp_in(x, w, b):
    return pl.pallas_call(
        _mlp_in_body,
        grid=(N_NODES // _R,),
        in_specs=[
            pl.BlockSpec((_R, D_IN), lambda i: (i, 0)),
            pl.BlockSpec((D_IN, D_H), lambda i: (0, 0)),
            pl.BlockSpec((1, D_H), lambda i: (0, 0)),
        ],
        out_specs=pl.BlockSpec((_R, D_A), lambda i: (i, 0)),
        out_shape=jax.ShapeDtypeStruct((N_NODES, D_A), jnp.float32),
    )(x, w, b)


def _combine_body(h_ref, agg_ref, ws_ref, bs_ref, wn_ref, bn_ref,
                  g_ref, be_ref, o_ref, *, last):
    h = h_ref[...][:, 0:D_H]
    a = agg_ref[...]
    self_f = jnp.dot(h, ws_ref[...], preferred_element_type=jnp.float32,
                     precision=_PREC) + bs_ref[...]
    cnt = jnp.maximum(a[:, D_H:D_H + 1], 1.0)
    agg = a[:, 0:D_H] / cnt
    neigh = jnp.dot(agg, wn_ref[...], preferred_element_type=jnp.float32,
                    precision=_PREC) + bn_ref[...]
    t = self_f + neigh
    mu = jnp.mean(t, axis=-1, keepdims=True)
    var = jnp.mean((t - mu) ** 2, axis=-1, keepdims=True)
    t = (t - mu) / jnp.sqrt(var + 1e-5) * g_ref[...] + be_ref[...]
    if not last:
        t = jnp.maximum(t, 0.0) + h
        t = jnp.concatenate(
            [t, jnp.ones((t.shape[0], D_A - D_H), jnp.float32)], axis=1)
    o_ref[...] = t


def _combine(h, agg, ws, bs, wn, bn, g, be, last):
    d_out = D_H if last else D_A
    return pl.pallas_call(
        functools.partial(_combine_body, last=last),
        grid=(N_NODES // _R,),
        in_specs=[
            pl.BlockSpec((_R, D_A), lambda i: (i, 0)),
            pl.BlockSpec((_R, D_A), lambda i: (i, 0)),
            pl.BlockSpec((D_H, D_H), lambda i: (0, 0)),
            pl.BlockSpec((1, D_H), lambda i: (0, 0)),
            pl.BlockSpec((D_H, D_H), lambda i: (0, 0)),
            pl.BlockSpec((1, D_H), lambda i: (0, 0)),
            pl.BlockSpec((1, D_H), lambda i: (0, 0)),
            pl.BlockSpec((1, D_H), lambda i: (0, 0)),
        ],
        out_specs=pl.BlockSpec((_R, d_out), lambda i: (i, 0)),
        out_shape=jax.ShapeDtypeStruct((N_NODES, d_out), jnp.float32),
    )(h, agg, ws, bs, wn, bn, g, be)


def _merge_halves(y):
    return jnp.concatenate([y[0, :HALF], y[1, :HALF]], axis=0)


def kernel(x, edge_index, W_in, b_in, Ws0, bs0, Wn0, bn0, g0, be0,
           Ws1, bs1, Wn1, bn1, g1, be1):
    src2 = edge_index[0].astype(jnp.int32).reshape(RT, G)
    dst2 = edge_index[1].astype(jnp.int32).reshape(RT, G)
    zeros80 = jnp.zeros((STRIPE, D_A), jnp.float32)

    h0 = _mlp_in(x, W_in, b_in.reshape(1, -1))

    agg0 = _merge_halves(_sc_agg(h0, src2, dst2, zeros80))
    h1 = _combine(h0, agg0, Ws0, bs0.reshape(1, -1), Wn0, bn0.reshape(1, -1),
                  g0.reshape(1, -1), be0.reshape(1, -1), last=False)

    agg1 = _merge_halves(_sc_agg(h1, src2, dst2, zeros80))
    out = _combine(h1, agg1, Ws1, bs1.reshape(1, -1), Wn1, bn1.reshape(1, -1),
                   g1.reshape(1, -1), be1.reshape(1, -1), last=True)
    return out


# speedup vs baseline: 1.3820x; 1.3820x over previous
import functools

import jax
import jax.numpy as jnp
from jax import lax
from jax.experimental import pallas as pl
from jax.experimental.pallas import tpu as pltpu
from jax.experimental.pallas import tpu_sc as plsc

N_NODES = 50000
N_EDGES = 800000
D_IN = 128
D_H = 64

NC = 2
NS = 16
HALF = N_NODES // NC
STRIPE = 1568
ROWS = NS * STRIPE
P = NC * ROWS
PAD = ROWS - HALF
TRASH = HALF
G = 80
RT = N_EDGES // G
NCH = RT // NS
KI = 7
KR = 5

_sc_mesh = plsc.VectorSubcoreMesh(core_axis_name="c", subcore_axis_name="s")
_sc_params = pltpu.CompilerParams(use_tc_tiling_on_sc=False)


def _localize(base, ldst_v, slot, g, voff=None):
    for q in range(g // 16):
        d = ldst_v[slot, pl.ds(q * 16, 16)]
        if voff is not None:
            d = d + voff
        m = (d >= base) & (d < base + HALF)
        ldst_v[slot, pl.ds(q * 16, 16)] = jnp.where(m, d - base, TRASH)


def _pad_src(src_v, slot):
    for q in range(G // 16):
        v = src_v[slot, pl.ds(q * 16, 16)]
        src_v[slot, pl.ds(q * 16, 16)] = jnp.where(v >= HALF, v + PAD, v)


def _make_agg():
    scratch = [
        pltpu.VMEM((KI, G), jnp.int32),
        pltpu.VMEM((KI, G), jnp.int32),
        pltpu.VMEM((KR, G, D_H), jnp.float32),
        pltpu.VMEM_SHARED((ROWS, D_H), jnp.float32),
        pltpu.SemaphoreType.DMA,
        pltpu.SemaphoreType.DMA,
        pltpu.SemaphoreType.DMA,
        pltpu.SemaphoreType.DMA,
    ]

    def body(h_hbm, src2, dst2, z64, agg_out,
             src_v, ldst_v, rows_v, acc_sh, sem_ia, sem_ib, sem_g, sem_s):
        c = lax.axis_index("c")
        s = lax.axis_index("s")
        base = c * HALF
        start = s * NCH

        def fire_idx(j, sem):
            slot = lax.rem(j, KI)
            pltpu.async_copy(src2.at[start + j], src_v.at[slot], sem)
            pltpu.async_copy(dst2.at[start + j], ldst_v.at[slot], sem)

        def drain_idx(j, sem):
            slot = lax.rem(j, KI)
            pltpu.make_async_copy(src2.at[0], src_v.at[slot], sem).wait()
            pltpu.make_async_copy(dst2.at[0], ldst_v.at[slot], sem).wait()

        def prep(j):
            slot = lax.rem(j, KI)
            _pad_src(src_v, slot)
            _localize(base, ldst_v, slot, G)

        def fire_gather(j):
            pltpu.async_copy(h_hbm.at[src_v.at[lax.rem(j, KI)]],
                             rows_v.at[lax.rem(j, KR)], sem_g)

        def wait_gather(j):
            pltpu.make_async_copy(h_hbm.at[src_v.at[lax.rem(j, KI)]],
                                  rows_v.at[lax.rem(j, KR)], sem_g).wait()

        def fire_scatter(j):
            pltpu.async_copy(rows_v.at[lax.rem(j, KR)],
                             acc_sh.at[ldst_v.at[lax.rem(j, KI)]],
                             sem_s, add=True)

        def wait_scatter(j):
            pltpu.make_async_copy(rows_v.at[lax.rem(j, KR)],
                                  acc_sh.at[ldst_v.at[lax.rem(j, KI)]],
                                  sem_s).wait()

        def isem(j):
            return sem_ia if j % 2 == 0 else sem_ib

        pltpu.sync_copy(z64, acc_sh.at[pl.ds(s * STRIPE, STRIPE)])
        plsc.subcore_barrier()
        for m in range(4):
            fire_idx(m, isem(m))
            drain_idx(m, isem(m))
            prep(m)
            fire_gather(m)
        fire_idx(4, sem_ia)
        fire_idx(5, sem_ib)

        def cu(j, par):
            wait_gather(j)
            fire_scatter(j)

            @pl.when(j >= 1)
            def _():
                wait_scatter(j - 1)

            @pl.when(j + 4 < NCH)
            def _():
                drain_idx(j + 4, isem(par + 4))
                prep(j + 4)
                fire_gather(j + 4)

            @pl.when(j + 6 < NCH)
            def _():
                fire_idx(j + 6, isem(par + 6))

        def pair_body(i, carry):
            cu(2 * i, 0)
            cu(2 * i + 1, 1)
            return carry

        lax.fori_loop(0, NCH // 2, pair_body, 0)
        cu(NCH - 1, (NCH - 1) % 2)
        wait_scatter(NCH - 1)

        plsc.subcore_barrier()
        pltpu.sync_copy(acc_sh.at[pl.ds(s * STRIPE, STRIPE)],
                        agg_out.at[c].at[pl.ds(s * STRIPE, STRIPE)])

    return pl.kernel(
        body,
        out_type=jax.ShapeDtypeStruct((NC, ROWS, D_H), jnp.float32),
        mesh=_sc_mesh,
        scratch_types=scratch,
        compiler_params=_sc_params,
    )


GC = 128
RTC = N_EDGES // GC
NCHC = 392


def _make_counts():
    scratch = [
        pltpu.VMEM((5, GC), jnp.int32),
        pltpu.VMEM((GC, 16), jnp.float32),
        pltpu.VMEM_SHARED((ROWS, 16), jnp.float32),
        pltpu.SemaphoreType.DMA,
        pltpu.SemaphoreType.DMA,
        pltpu.SemaphoreType.DMA,
    ]

    def body(dst3, z16, ones_hbm, cnt_out, ldst_v, ones_v, cnt_sh,
             sem_ia, sem_ib, sem_s):
        c = lax.axis_index("c")
        s = lax.axis_index("s")
        base = c * HALF
        start = s * (RTC // NS) + jnp.minimum(s, RTC % NS)
        nrows = (RTC // NS) + jnp.where(s < RTC % NS, 1, 0)

        def fire_idx(j, sem):
            r = jnp.minimum(start + j, RTC - 1)
            pltpu.async_copy(dst3.at[r], ldst_v.at[lax.rem(j, 5)], sem)

        def drain_idx(j, sem):
            pltpu.make_async_copy(dst3.at[0], ldst_v.at[lax.rem(j, 5)],
                                  sem).wait()

        def fire_scatter(j):
            pltpu.async_copy(ones_v, cnt_sh.at[ldst_v.at[lax.rem(j, 5)]],
                             sem_s, add=True)

        def wait_scatter(j):
            pltpu.make_async_copy(ones_v, cnt_sh.at[ldst_v.at[lax.rem(j, 5)]],
                                  sem_s).wait()

        def isem(j):
            return sem_ia if j % 2 == 0 else sem_ib

        pltpu.sync_copy(z16, cnt_sh.at[pl.ds(s * STRIPE, STRIPE)])
        pltpu.sync_copy(ones_hbm, ones_v)
        plsc.subcore_barrier()
        fire_idx(0, sem_ia)
        fire_idx(1, sem_ib)

        def cu(j, par):
            drain_idx(j, isem(par))
            voff = jnp.where(j < nrows, 0, N_NODES)
            _localize(base, ldst_v, lax.rem(j, 5), GC, voff)
            fire_scatter(j)

            @pl.when(j >= 1)
            def _():
                wait_scatter(j - 1)

            @pl.when(j + 2 < NCHC)
            def _():
                fire_idx(j + 2, isem(par))

        def pair_body(i, carry):
            cu(2 * i, 0)
            cu(2 * i + 1, 1)
            return carry

        lax.fori_loop(0, NCHC // 2, pair_body, 0)
        wait_scatter(NCHC - 1)

        plsc.subcore_barrier()
        pltpu.sync_copy(cnt_sh.at[pl.ds(s * STRIPE, STRIPE)],
                        cnt_out.at[c].at[pl.ds(s * STRIPE, STRIPE)])

    return pl.kernel(
        body,
        out_type=jax.ShapeDtypeStruct((NC, ROWS, 16), jnp.float32),
        mesh=_sc_mesh,
        scratch_types=scratch,
        compiler_params=_sc_params,
    )


_sc_agg = _make_agg()
_sc_counts = _make_counts()


_R = 784
_PREC = lax.Precision.HIGHEST


def _mlp_in_body(x_ref, w_ref, b_ref, o_ref):
    o_ref[...] = jnp.maximum(
        jnp.dot(x_ref[...], w_ref[...], preferred_element_type=jnp.float32,
                precision=_PREC) + b_ref[...], 0.0)


def _mlp_in(x, w, b):
    return pl.pallas_call(
        _mlp_in_body,
        grid=(P // _R,),
        in_specs=[
            pl.BlockSpec((_R, D_IN), lambda i: (i, 0)),
            pl.BlockSpec((D_IN, D_H), lambda i: (0, 0)),
            pl.BlockSpec((1, D_H), lambda i: (0, 0)),
        ],
        out_specs=pl.BlockSpec((_R, D_H), lambda i: (i, 0)),
        out_shape=jax.ShapeDtypeStruct((P, D_H), jnp.float32),
    )(x, w, b)


def _combine_body(h_ref, agg_ref, cnt_ref, ws_ref, bs_ref, wn_ref, bn_ref,
                  g_ref, be_ref, o_ref, *, last):
    h = h_ref[...]
    self_f = jnp.dot(h, ws_ref[...], preferred_element_type=jnp.float32,
                     precision=_PREC) + bs_ref[...]
    cnt = jnp.maximum(cnt_ref[...][:, 0:1], 1.0)
    agg = agg_ref[...] / cnt
    neigh = jnp.dot(agg, wn_ref[...], preferred_element_type=jnp.float32,
                    precision=_PREC) + bn_ref[...]
    t = self_f + neigh
    mu = jnp.mean(t, axis=-1, keepdims=True)
    var = jnp.mean((t - mu) ** 2, axis=-1, keepdims=True)
    t = (t - mu) / jnp.sqrt(var + 1e-5) * g_ref[...] + be_ref[...]
    if not last:
        t = jnp.maximum(t, 0.0) + h
    o_ref[...] = t


def _combine(h, agg, cnt, ws, bs, wn, bn, g, be, last):
    return pl.pallas_call(
        functools.partial(_combine_body, last=last),
        grid=(P // _R,),
        in_specs=[
            pl.BlockSpec((_R, D_H), lambda i: (i, 0)),
            pl.BlockSpec((_R, D_H), lambda i: (i, 0)),
            pl.BlockSpec((_R, 16), lambda i: (i, 0)),
            pl.BlockSpec((D_H, D_H), lambda i: (0, 0)),
            pl.BlockSpec((1, D_H), lambda i: (0, 0)),
            pl.BlockSpec((D_H, D_H), lambda i: (0, 0)),
            pl.BlockSpec((1, D_H), lambda i: (0, 0)),
            pl.BlockSpec((1, D_H), lambda i: (0, 0)),
            pl.BlockSpec((1, D_H), lambda i: (0, 0)),
        ],
        out_specs=pl.BlockSpec((_R, D_H), lambda i: (i, 0)),
        out_shape=jax.ShapeDtypeStruct((P, D_H), jnp.float32),
    )(h, agg, cnt, ws, bs, wn, bn, g, be)


def kernel(x, edge_index, W_in, b_in, Ws0, bs0, Wn0, bn0, g0, be0,
           Ws1, bs1, Wn1, bn1, g1, be1):
    src2 = edge_index[0].astype(jnp.int32).reshape(RT, G)
    dst = edge_index[1].astype(jnp.int32)
    dst2 = dst.reshape(RT, G)
    dst3 = dst.reshape(RTC, GC)
    zeros64 = jnp.zeros((STRIPE, D_H), jnp.float32)
    zeros16 = jnp.zeros((STRIPE, 16), jnp.float32)
    ones16 = jnp.ones((GC, 16), jnp.float32)

    xp = jnp.zeros((P, D_IN), x.dtype)
    xp = xp.at[0:HALF].set(x[0:HALF])
    xp = xp.at[ROWS:ROWS + HALF].set(x[HALF:])

    h0 = _mlp_in(xp, W_in, b_in.reshape(1, -1))

    cnt = _sc_counts(dst3, zeros16, ones16).reshape(P, 16)
    agg0 = _sc_agg(h0, src2, dst2, zeros64).reshape(P, D_H)
    h1 = _combine(h0, agg0, cnt, Ws0, bs0.reshape(1, -1), Wn0, bn0.reshape(1, -1),
                  g0.reshape(1, -1), be0.reshape(1, -1), last=False)

    agg1 = _sc_agg(h1, src2, dst2, zeros64).reshape(P, D_H)
    out = _combine(h1, agg1, cnt, Ws1, bs1.reshape(1, -1), Wn1, bn1.reshape(1, -1),
                   g1.reshape(1, -1), be1.reshape(1, -1), last=True)
    return jnp.concatenate([out[0:HALF], out[ROWS:ROWS + HALF]], axis=0)


# speedup vs baseline: 1.4522x; 1.0508x over previous
import functools

import jax
import jax.numpy as jnp
from jax import lax
from jax.experimental import pallas as pl
from jax.experimental.pallas import tpu as pltpu
from jax.experimental.pallas import tpu_sc as plsc

N_NODES = 50000
N_EDGES = 800000
D_IN = 128
D_H = 64

NC = 2
NS = 16
HALF = N_NODES // NC
STRIPE = 1568
ROWS = NS * STRIPE
P = NC * ROWS
PAD = ROWS - HALF
TRASH = HALF
G = 80
RT = N_EDGES // G
NCH = RT // NS
KI = 7
KR = 5

_sc_mesh = plsc.VectorSubcoreMesh(core_axis_name="c", subcore_axis_name="s")
_sc_params = pltpu.CompilerParams(use_tc_tiling_on_sc=False)


def _localize(base, ldst_v, slot, g, voff=None):
    for q in range(g // 16):
        d = ldst_v[slot, pl.ds(q * 16, 16)]
        if voff is not None:
            d = d + voff
        m = (d >= base) & (d < base + HALF)
        ldst_v[slot, pl.ds(q * 16, 16)] = jnp.where(m, d - base, TRASH)


def _make_agg():
    scratch = [
        pltpu.VMEM((KI, G), jnp.int32),
        pltpu.VMEM((KI, G), jnp.int32),
        pltpu.VMEM((KR, G, D_H), jnp.float32),
        pltpu.VMEM_SHARED((ROWS, D_H), jnp.float32),
        pltpu.SemaphoreType.DMA,
        pltpu.SemaphoreType.DMA,
        pltpu.SemaphoreType.DMA,
        pltpu.SemaphoreType.DMA,
    ]

    def body(h_hbm, src2, dst2, z64, agg_out,
             src_v, ldst_v, rows_v, acc_sh, sem_ia, sem_ib, sem_g, sem_s):
        c = lax.axis_index("c")
        s = lax.axis_index("s")
        base = c * HALF
        start = s * NCH

        def fire_idx(j, sem):
            slot = lax.rem(j, KI)
            pltpu.async_copy(src2.at[start + j], src_v.at[slot], sem)
            pltpu.async_copy(dst2.at[start + j], ldst_v.at[slot], sem)

        def drain_idx(j, sem):
            slot = lax.rem(j, KI)
            pltpu.make_async_copy(src2.at[0], src_v.at[slot], sem).wait()
            pltpu.make_async_copy(dst2.at[0], ldst_v.at[slot], sem).wait()

        def prep(j):
            _localize(base, ldst_v, lax.rem(j, KI), G)

        def fire_gather(j):
            pltpu.async_copy(h_hbm.at[src_v.at[lax.rem(j, KI)]],
                             rows_v.at[lax.rem(j, KR)], sem_g)

        def wait_gather(j):
            pltpu.make_async_copy(h_hbm.at[src_v.at[lax.rem(j, KI)]],
                                  rows_v.at[lax.rem(j, KR)], sem_g).wait()

        def fire_scatter(j):
            pltpu.async_copy(rows_v.at[lax.rem(j, KR)],
                             acc_sh.at[ldst_v.at[lax.rem(j, KI)]],
                             sem_s, add=True)

        def wait_scatter(j):
            pltpu.make_async_copy(rows_v.at[lax.rem(j, KR)],
                                  acc_sh.at[ldst_v.at[lax.rem(j, KI)]],
                                  sem_s).wait()

        def isem(j):
            return sem_ia if j % 2 == 0 else sem_ib

        pltpu.sync_copy(z64, acc_sh.at[pl.ds(s * STRIPE, STRIPE)])
        plsc.subcore_barrier()
        for m in range(4):
            fire_idx(m, isem(m))
            drain_idx(m, isem(m))
            prep(m)
            fire_gather(m)
        fire_idx(4, sem_ia)
        fire_idx(5, sem_ib)

        def cu(j, par):
            wait_gather(j)
            fire_scatter(j)

            @pl.when(j >= 1)
            def _():
                wait_scatter(j - 1)

            @pl.when(j + 4 < NCH)
            def _():
                drain_idx(j + 4, isem(par + 4))
                prep(j + 4)
                fire_gather(j + 4)

            @pl.when(j + 6 < NCH)
            def _():
                fire_idx(j + 6, isem(par + 6))

        def pair_body(i, carry):
            cu(2 * i, 0)
            cu(2 * i + 1, 1)
            return carry

        lax.fori_loop(0, NCH // 2, pair_body, 0)
        cu(NCH - 1, (NCH - 1) % 2)
        wait_scatter(NCH - 1)

        plsc.subcore_barrier()
        pltpu.sync_copy(acc_sh.at[pl.ds(s * STRIPE, STRIPE)],
                        agg_out.at[c].at[pl.ds(s * STRIPE, STRIPE)])

    return pl.kernel(
        body,
        out_type=jax.ShapeDtypeStruct((NC, ROWS, D_H), jnp.float32),
        mesh=_sc_mesh,
        scratch_types=scratch,
        compiler_params=_sc_params,
    )


GC = 128
RTC = N_EDGES // GC
NCHC = 392


def _make_counts():
    scratch = [
        pltpu.VMEM((5, GC), jnp.int32),
        pltpu.VMEM((GC, 16), jnp.float32),
        pltpu.VMEM_SHARED((ROWS, 16), jnp.float32),
        pltpu.SemaphoreType.DMA,
        pltpu.SemaphoreType.DMA,
        pltpu.SemaphoreType.DMA,
    ]

    def body(dst3, z16, ones_hbm, cnt_out, ldst_v, ones_v, cnt_sh,
             sem_ia, sem_ib, sem_s):
        c = lax.axis_index("c")
        s = lax.axis_index("s")
        base = c * HALF
        start = s * (RTC // NS) + jnp.minimum(s, RTC % NS)
        nrows = (RTC // NS) + jnp.where(s < RTC % NS, 1, 0)

        def fire_idx(j, sem):
            r = jnp.minimum(start + j, RTC - 1)
            pltpu.async_copy(dst3.at[r], ldst_v.at[lax.rem(j, 5)], sem)

        def drain_idx(j, sem):
            pltpu.make_async_copy(dst3.at[0], ldst_v.at[lax.rem(j, 5)],
                                  sem).wait()

        def fire_scatter(j):
            pltpu.async_copy(ones_v, cnt_sh.at[ldst_v.at[lax.rem(j, 5)]],
                             sem_s, add=True)

        def wait_scatter(j):
            pltpu.make_async_copy(ones_v, cnt_sh.at[ldst_v.at[lax.rem(j, 5)]],
                                  sem_s).wait()

        def isem(j):
            return sem_ia if j % 2 == 0 else sem_ib

        pltpu.sync_copy(z16, cnt_sh.at[pl.ds(s * STRIPE, STRIPE)])
        pltpu.sync_copy(ones_hbm, ones_v)
        plsc.subcore_barrier()
        fire_idx(0, sem_ia)
        fire_idx(1, sem_ib)

        def cu(j, par):
            drain_idx(j, isem(par))
            voff = jnp.where(j < nrows, 0, N_NODES)
            _localize(base, ldst_v, lax.rem(j, 5), GC, voff)
            fire_scatter(j)

            @pl.when(j >= 1)
            def _():
                wait_scatter(j - 1)

            @pl.when(j + 2 < NCHC)
            def _():
                fire_idx(j + 2, isem(par))

        def pair_body(i, carry):
            cu(2 * i, 0)
            cu(2 * i + 1, 1)
            return carry

        lax.fori_loop(0, NCHC // 2, pair_body, 0)
        wait_scatter(NCHC - 1)

        plsc.subcore_barrier()
        pltpu.sync_copy(cnt_sh.at[pl.ds(s * STRIPE, STRIPE)],
                        cnt_out.at[c].at[pl.ds(s * STRIPE, STRIPE)])

    return pl.kernel(
        body,
        out_type=jax.ShapeDtypeStruct((NC, ROWS, 16), jnp.float32),
        mesh=_sc_mesh,
        scratch_types=scratch,
        compiler_params=_sc_params,
    )


_sc_agg = _make_agg()
_sc_counts = _make_counts()


_R = 1000
_PREC = lax.Precision.HIGHEST


def _mlp_in_body(x_ref, w_ref, b_ref, o_ref):
    o_ref[...] = jnp.maximum(
        jnp.dot(x_ref[...], w_ref[...], preferred_element_type=jnp.float32,
                precision=_PREC) + b_ref[...], 0.0)


def _mlp_in(x, w, b):
    return pl.pallas_call(
        _mlp_in_body,
        grid=(N_NODES // _R,),
        in_specs=[
            pl.BlockSpec((_R, D_IN), lambda i: (i, 0)),
            pl.BlockSpec((D_IN, D_H), lambda i: (0, 0)),
            pl.BlockSpec((1, D_H), lambda i: (0, 0)),
        ],
        out_specs=pl.BlockSpec((_R, D_H), lambda i: (i, 0)),
        out_shape=jax.ShapeDtypeStruct((N_NODES, D_H), jnp.float32),
    )(x, w, b)


def _combine_body(h_ref, agg_ref, cnt_ref, ws_ref, bs_ref, wn_ref, bn_ref,
                  g_ref, be_ref, o_ref, *, last):
    h = h_ref[...]
    self_f = jnp.dot(h, ws_ref[...], preferred_element_type=jnp.float32,
                     precision=_PREC) + bs_ref[...]
    cnt = jnp.maximum(cnt_ref[0][:, 0:1], 1.0)
    agg = agg_ref[0] / cnt
    neigh = jnp.dot(agg, wn_ref[...], preferred_element_type=jnp.float32,
                    precision=_PREC) + bn_ref[...]
    t = self_f + neigh
    mu = jnp.mean(t, axis=-1, keepdims=True)
    var = jnp.mean((t - mu) ** 2, axis=-1, keepdims=True)
    t = (t - mu) / jnp.sqrt(var + 1e-5) * g_ref[...] + be_ref[...]
    if not last:
        t = jnp.maximum(t, 0.0) + h
    o_ref[...] = t


_HB = HALF // _R


def _combine(h, agg, cnt, ws, bs, wn, bn, g, be, last):
    return pl.pallas_call(
        functools.partial(_combine_body, last=last),
        grid=(N_NODES // _R,),
        in_specs=[
            pl.BlockSpec((_R, D_H), lambda i: (i, 0)),
            pl.BlockSpec((1, _R, D_H), lambda i: (i // _HB, i % _HB, 0)),
            pl.BlockSpec((1, _R, 16), lambda i: (i // _HB, i % _HB, 0)),
            pl.BlockSpec((D_H, D_H), lambda i: (0, 0)),
            pl.BlockSpec((1, D_H), lambda i: (0, 0)),
            pl.BlockSpec((D_H, D_H), lambda i: (0, 0)),
            pl.BlockSpec((1, D_H), lambda i: (0, 0)),
            pl.BlockSpec((1, D_H), lambda i: (0, 0)),
            pl.BlockSpec((1, D_H), lambda i: (0, 0)),
        ],
        out_specs=pl.BlockSpec((_R, D_H), lambda i: (i, 0)),
        out_shape=jax.ShapeDtypeStruct((N_NODES, D_H), jnp.float32),
    )(h, agg, cnt, ws, bs, wn, bn, g, be)


def kernel(x, edge_index, W_in, b_in, Ws0, bs0, Wn0, bn0, g0, be0,
           Ws1, bs1, Wn1, bn1, g1, be1):
    src2 = edge_index[0].astype(jnp.int32).reshape(RT, G)
    dst = edge_index[1].astype(jnp.int32)
    dst2 = dst.reshape(RT, G)
    dst3 = dst.reshape(RTC, GC)
    zeros64 = jnp.zeros((STRIPE, D_H), jnp.float32)
    zeros16 = jnp.zeros((STRIPE, 16), jnp.float32)
    ones16 = jnp.ones((GC, 16), jnp.float32)

    h0 = _mlp_in(x, W_in, b_in.reshape(1, -1))

    cnt = _sc_counts(dst3, zeros16, ones16)
    agg0 = _sc_agg(h0, src2, dst2, zeros64)
    h1 = _combine(h0, agg0, cnt, Ws0, bs0.reshape(1, -1), Wn0, bn0.reshape(1, -1),
                  g0.reshape(1, -1), be0.reshape(1, -1), last=False)

    agg1 = _sc_agg(h1, src2, dst2, zeros64)
    out = _combine(h1, agg1, cnt, Ws1, bs1.reshape(1, -1), Wn1, bn1.reshape(1, -1),
                   g1.reshape(1, -1), be1.reshape(1, -1), last=True)
    return out


# speedup vs baseline: 1.4650x; 1.0088x over previous
import functools

import jax
import jax.numpy as jnp
from jax import lax
from jax.experimental import pallas as pl
from jax.experimental.pallas import tpu as pltpu
from jax.experimental.pallas import tpu_sc as plsc

N_NODES = 50000
N_EDGES = 800000
D_IN = 128
D_H = 64

NC = 2
NS = 16
HALF = N_NODES // NC
STRIPE = 1568
ROWS = NS * STRIPE
P = NC * ROWS
PAD = ROWS - HALF
TRASH = HALF
G = 80
RT = N_EDGES // G
NCH = RT // NS
KI = 7
KR = 5

_sc_mesh = plsc.VectorSubcoreMesh(core_axis_name="c", subcore_axis_name="s")
_sc_params = pltpu.CompilerParams(use_tc_tiling_on_sc=False)


def _localize(base, ldst_v, slot, g, voff=None):
    for q in range(g // 16):
        d = ldst_v[slot, pl.ds(q * 16, 16)]
        if voff is not None:
            d = d + voff
        m = (d >= base) & (d < base + HALF)
        ldst_v[slot, pl.ds(q * 16, 16)] = jnp.where(m, d - base, TRASH)


def _make_agg():
    scratch = [
        pltpu.VMEM((KI, G), jnp.int32),
        pltpu.VMEM((KI, G), jnp.int32),
        pltpu.VMEM((KR, G, D_H), jnp.float32),
        pltpu.VMEM_SHARED((ROWS, D_H), jnp.float32),
        pltpu.SemaphoreType.DMA,
        pltpu.SemaphoreType.DMA,
        pltpu.SemaphoreType.DMA,
        pltpu.SemaphoreType.DMA,
    ]

    def body(h_hbm, src2, dst2, z64, agg_out,
             src_v, ldst_v, rows_v, acc_sh, sem_ia, sem_ib, sem_g, sem_s):
        c = lax.axis_index("c")
        s = lax.axis_index("s")
        base = c * HALF
        start = s * NCH

        def fire_idx(j, sem):
            slot = lax.rem(j, KI)
            pltpu.async_copy(src2.at[start + j], src_v.at[slot], sem)
            pltpu.async_copy(dst2.at[start + j], ldst_v.at[slot], sem)

        def drain_idx(j, sem):
            slot = lax.rem(j, KI)
            pltpu.make_async_copy(src2.at[0], src_v.at[slot], sem).wait()
            pltpu.make_async_copy(dst2.at[0], ldst_v.at[slot], sem).wait()

        def prep(j):
            _localize(base, ldst_v, lax.rem(j, KI), G)

        def fire_gather(j):
            pltpu.async_copy(h_hbm.at[src_v.at[lax.rem(j, KI)]],
                             rows_v.at[lax.rem(j, KR)], sem_g)

        def wait_gather(j):
            pltpu.make_async_copy(h_hbm.at[src_v.at[lax.rem(j, KI)]],
                                  rows_v.at[lax.rem(j, KR)], sem_g).wait()

        def fire_scatter(j):
            pltpu.async_copy(rows_v.at[lax.rem(j, KR)],
                             acc_sh.at[ldst_v.at[lax.rem(j, KI)]],
                             sem_s, add=True)

        def wait_scatter(j):
            pltpu.make_async_copy(rows_v.at[lax.rem(j, KR)],
                                  acc_sh.at[ldst_v.at[lax.rem(j, KI)]],
                                  sem_s).wait()

        def isem(j):
            return sem_ia if j % 2 == 0 else sem_ib

        pltpu.sync_copy(z64, acc_sh.at[pl.ds(s * STRIPE, STRIPE)])
        plsc.subcore_barrier()
        for m in range(4):
            fire_idx(m, isem(m))
            drain_idx(m, isem(m))
            prep(m)
            fire_gather(m)
        fire_idx(4, sem_ia)
        fire_idx(5, sem_ib)

        def cu(j, par):
            wait_gather(j)
            fire_scatter(j)

            @pl.when(j >= 1)
            def _():
                wait_scatter(j - 1)

            @pl.when(j + 4 < NCH)
            def _():
                drain_idx(j + 4, isem(par + 4))
                prep(j + 4)
                fire_gather(j + 4)

            @pl.when(j + 6 < NCH)
            def _():
                fire_idx(j + 6, isem(par + 6))

        def pair_body(i, carry):
            cu(2 * i, 0)
            cu(2 * i + 1, 1)
            return carry

        lax.fori_loop(0, NCH // 2, pair_body, 0)
        cu(NCH - 1, (NCH - 1) % 2)
        wait_scatter(NCH - 1)

        plsc.subcore_barrier()
        pltpu.sync_copy(acc_sh.at[pl.ds(s * STRIPE, STRIPE)],
                        agg_out.at[c].at[pl.ds(s * STRIPE, STRIPE)])

    return pl.kernel(
        body,
        out_type=jax.ShapeDtypeStruct((NC, ROWS, D_H), jnp.float32),
        mesh=_sc_mesh,
        scratch_types=scratch,
        compiler_params=_sc_params,
    )


GC = 128
RTC = N_EDGES // GC
NCHC = 392


def _make_counts():
    scratch = [
        pltpu.VMEM((5, GC), jnp.int32),
        pltpu.VMEM((GC, 16), jnp.float32),
        pltpu.VMEM_SHARED((ROWS, 16), jnp.float32),
        pltpu.SemaphoreType.DMA,
        pltpu.SemaphoreType.DMA,
        pltpu.SemaphoreType.DMA,
    ]

    def body(dst3, z16, ones_hbm, cnt_out, ldst_v, ones_v, cnt_sh,
             sem_ia, sem_ib, sem_s):
        c = lax.axis_index("c")
        s = lax.axis_index("s")
        base = c * HALF
        start = s * (RTC // NS) + jnp.minimum(s, RTC % NS)
        nrows = (RTC // NS) + jnp.where(s < RTC % NS, 1, 0)

        def fire_idx(j, sem):
            r = jnp.minimum(start + j, RTC - 1)
            pltpu.async_copy(dst3.at[r], ldst_v.at[lax.rem(j, 5)], sem)

        def drain_idx(j, sem):
            pltpu.make_async_copy(dst3.at[0], ldst_v.at[lax.rem(j, 5)],
                                  sem).wait()

        def fire_scatter(j):
            pltpu.async_copy(ones_v, cnt_sh.at[ldst_v.at[lax.rem(j, 5)]],
                             sem_s, add=True)

        def wait_scatter(j):
            pltpu.make_async_copy(ones_v, cnt_sh.at[ldst_v.at[lax.rem(j, 5)]],
                                  sem_s).wait()

        def isem(j):
            return sem_ia if j % 2 == 0 else sem_ib

        pltpu.sync_copy(z16, cnt_sh.at[pl.ds(s * STRIPE, STRIPE)])
        pltpu.sync_copy(ones_hbm, ones_v)
        plsc.subcore_barrier()
        fire_idx(0, sem_ia)
        fire_idx(1, sem_ib)

        def cu(j, par):
            drain_idx(j, isem(par))
            voff = jnp.where(j < nrows, 0, N_NODES)
            _localize(base, ldst_v, lax.rem(j, 5), GC, voff)
            fire_scatter(j)

            @pl.when(j >= 1)
            def _():
                wait_scatter(j - 1)

            @pl.when(j + 2 < NCHC)
            def _():
                fire_idx(j + 2, isem(par))

        def pair_body(i, carry):
            cu(2 * i, 0)
            cu(2 * i + 1, 1)
            return carry

        lax.fori_loop(0, NCHC // 2, pair_body, 0)
        wait_scatter(NCHC - 1)

        plsc.subcore_barrier()
        pltpu.sync_copy(cnt_sh.at[pl.ds(s * STRIPE, STRIPE)],
                        cnt_out.at[c].at[pl.ds(s * STRIPE, STRIPE)])

    return pl.kernel(
        body,
        out_type=jax.ShapeDtypeStruct((NC, ROWS, 16), jnp.float32),
        mesh=_sc_mesh,
        scratch_types=scratch,
        compiler_params=_sc_params,
    )


_sc_agg = _make_agg()
_sc_counts = _make_counts()


_R = 5000
_PREC = lax.Precision.HIGHEST


def _mlp_in_body(x_ref, w_ref, b_ref, o_ref):
    o_ref[...] = jnp.maximum(
        jnp.dot(x_ref[...], w_ref[...], preferred_element_type=jnp.float32,
                precision=_PREC) + b_ref[...], 0.0)


def _mlp_in(x, w, b):
    return pl.pallas_call(
        _mlp_in_body,
        grid=(N_NODES // _R,),
        in_specs=[
            pl.BlockSpec((_R, D_IN), lambda i: (i, 0)),
            pl.BlockSpec((D_IN, D_H), lambda i: (0, 0)),
            pl.BlockSpec((1, D_H), lambda i: (0, 0)),
        ],
        out_specs=pl.BlockSpec((_R, D_H), lambda i: (i, 0)),
        out_shape=jax.ShapeDtypeStruct((N_NODES, D_H), jnp.float32),
    )(x, w, b)


def _combine_body(h_ref, agg_ref, cnt_ref, ws_ref, bs_ref, wn_ref, bn_ref,
                  g_ref, be_ref, o_ref, *, last):
    h = h_ref[...]
    self_f = jnp.dot(h, ws_ref[...], preferred_element_type=jnp.float32,
                     precision=_PREC) + bs_ref[...]
    cnt = jnp.maximum(cnt_ref[0][:, 0:1], 1.0)
    agg = agg_ref[0] / cnt
    neigh = jnp.dot(agg, wn_ref[...], preferred_element_type=jnp.float32,
                    precision=_PREC) + bn_ref[...]
    t = self_f + neigh
    mu = jnp.mean(t, axis=-1, keepdims=True)
    var = jnp.mean((t - mu) ** 2, axis=-1, keepdims=True)
    t = (t - mu) / jnp.sqrt(var + 1e-5) * g_ref[...] + be_ref[...]
    if not last:
        t = jnp.maximum(t, 0.0) + h
    o_ref[...] = t


_HB = HALF // _R


def _combine(h, agg, cnt, ws, bs, wn, bn, g, be, last):
    return pl.pallas_call(
        functools.partial(_combine_body, last=last),
        grid=(N_NODES // _R,),
        in_specs=[
            pl.BlockSpec((_R, D_H), lambda i: (i, 0)),
            pl.BlockSpec((1, _R, D_H), lambda i: (i // _HB, i % _HB, 0)),
            pl.BlockSpec((1, _R, 16), lambda i: (i // _HB, i % _HB, 0)),
            pl.BlockSpec((D_H, D_H), lambda i: (0, 0)),
            pl.BlockSpec((1, D_H), lambda i: (0, 0)),
            pl.BlockSpec((D_H, D_H), lambda i: (0, 0)),
            pl.BlockSpec((1, D_H), lambda i: (0, 0)),
            pl.BlockSpec((1, D_H), lambda i: (0, 0)),
            pl.BlockSpec((1, D_H), lambda i: (0, 0)),
        ],
        out_specs=pl.BlockSpec((_R, D_H), lambda i: (i, 0)),
        out_shape=jax.ShapeDtypeStruct((N_NODES, D_H), jnp.float32),
    )(h, agg, cnt, ws, bs, wn, bn, g, be)


def kernel(x, edge_index, W_in, b_in, Ws0, bs0, Wn0, bn0, g0, be0,
           Ws1, bs1, Wn1, bn1, g1, be1):
    src2 = edge_index[0].astype(jnp.int32).reshape(RT, G)
    dst = edge_index[1].astype(jnp.int32)
    dst2 = dst.reshape(RT, G)
    dst3 = dst.reshape(RTC, GC)
    zeros64 = jnp.zeros((STRIPE, D_H), jnp.float32)
    zeros16 = jnp.zeros((STRIPE, 16), jnp.float32)
    ones16 = jnp.ones((GC, 16), jnp.float32)

    h0 = _mlp_in(x, W_in, b_in.reshape(1, -1))

    cnt = _sc_counts(dst3, zeros16, ones16)
    agg0 = _sc_agg(h0, src2, dst2, zeros64)
    h1 = _combine(h0, agg0, cnt, Ws0, bs0.reshape(1, -1), Wn0, bn0.reshape(1, -1),
                  g0.reshape(1, -1), be0.reshape(1, -1), last=False)

    agg1 = _sc_agg(h1, src2, dst2, zeros64)
    out = _combine(h1, agg1, cnt, Ws1, bs1.reshape(1, -1), Wn1, bn1.reshape(1, -1),
                   g1.reshape(1, -1), be1.reshape(1, -1), last=True)
    return out
